# baseline (speedup 1.0000x reference)
import functools

import jax
import jax.numpy as jnp
from jax import lax
from jax.experimental import pallas as pl
from jax.experimental.pallas import tpu as pltpu
from jax.experimental.pallas import tpu_sc as plsc

N_SC = 2
N_TILE = 16
NW = N_SC * N_TILE
B_EDGE = 128
DEG_W = 8


def _sc_mesh():
    return plsc.VectorSubcoreMesh(core_axis_name="c", subcore_axis_name="s")


def _sc_segsum(g, srcp, dstp, zrows, nacc, rpt, nb0, nb1):
    n_nodes, d = g.shape
    dt = g.dtype
    nb = srcp.shape[1]

    @functools.partial(
        pl.kernel,
        mesh=_sc_mesh(),
        compiler_params=pltpu.CompilerParams(use_tc_tiling_on_sc=False),
        out_type=jax.ShapeDtypeStruct((N_SC, nacc, d), dt),
        scratch_types=[
            pltpu.VMEM((nb, B_EDGE), jnp.int32),
            pltpu.VMEM((nb, B_EDGE), jnp.int32),
            [pltpu.VMEM((B_EDGE, d), dt)] * 8,
            pltpu.VMEM_SHARED((nacc, d), dt),
            [pltpu.SemaphoreType.DMA] * 8,
            [pltpu.SemaphoreType.DMA] * 8,
        ],
    )
    def seg(g_hbm, src_hbm, dst_hbm, z_hbm, out_hbm,
            src_v, dst_v, bufs, acc, gsems, ssems):
        cid = lax.axis_index("c")
        sid = lax.axis_index("s")
        wid = cid * N_TILE + sid

        pltpu.sync_copy(src_hbm.at[wid], src_v)
        pltpu.sync_copy(dst_hbm.at[wid], dst_v)
        pltpu.sync_copy(z_hbm, acc.at[pl.ds(sid * rpt, rpt)])
        plsc.subcore_barrier()

        def gather_start(j, b):
            pltpu.make_async_copy(g_hbm.at[src_v.at[j]], bufs[b], gsems[b]).start()

        def gather_wait(b):
            pltpu.make_async_copy(g_hbm.at[src_v.at[0]], bufs[b], gsems[b]).wait()

        def scatter_start(j, b):
            pltpu.make_async_copy(bufs[b], acc.at[dst_v.at[j]], ssems[b]).start(add=True)

        def scatter_wait(b):
            pltpu.make_async_copy(bufs[b], acc.at[dst_v.at[0]], ssems[b]).wait()

        nbuf = 8
        look = nbuf // 2

        def run_ring(nbc):
            for k in range(look):
                gather_start(k, k)

            def body(i, _):
                for b in range(nbuf):
                    j = nbuf * i + b
                    gather_wait(b)
                    scatter_start(j, b)
                    nxt = (b + look) % nbuf

                    @pl.when(j + look < nbc)
                    def _():
                        @pl.when(j >= look)
                        def _():
                            scatter_wait(nxt)

                        gather_start(j + look, nxt)
                return 0

            lax.fori_loop(0, nbc // nbuf, body, 0)
            for k in range(nbuf):
                scatter_wait(k)

        @pl.when(cid == 0)
        def _():
            run_ring(nb0)

        @pl.when(cid == 1)
        def _():
            run_ring(nb1)
        plsc.subcore_barrier()
        pltpu.sync_copy(acc.at[pl.ds(sid * rpt, rpt)],
                        out_hbm.at[cid, pl.ds(sid * rpt, rpt)])

    return seg(g, srcp, dstp, zrows)


def _sc_degree(dstp, ones_rows, zrows, nacc, rpt, nb0, nb1):
    nb = dstp.shape[1]

    @functools.partial(
        pl.kernel,
        mesh=_sc_mesh(),
        compiler_params=pltpu.CompilerParams(use_tc_tiling_on_sc=False),
        out_type=jax.ShapeDtypeStruct((N_SC, nacc, DEG_W), jnp.float32),
        scratch_types=[
            pltpu.VMEM((nb, B_EDGE), jnp.int32),
            pltpu.VMEM((B_EDGE, DEG_W), jnp.float32),
            pltpu.VMEM_SHARED((nacc, DEG_W), jnp.float32),
        ],
    )
    def degk(dst_hbm, ones_hbm, z_hbm, out_hbm, dst_v, ones_v, acc):
        cid = lax.axis_index("c")
        sid = lax.axis_index("s")
        wid = cid * N_TILE + sid

        pltpu.sync_copy(dst_hbm.at[wid], dst_v)
        pltpu.sync_copy(ones_hbm, ones_v)
        pltpu.sync_copy(z_hbm, acc.at[pl.ds(sid * rpt, rpt)])
        plsc.subcore_barrier()

        def body(j, _):
            pltpu.sync_copy(ones_v, acc.at[dst_v.at[j]], add=True)
            return 0

        @pl.when(cid == 0)
        def _():
            lax.fori_loop(0, nb0, body, 0)

        @pl.when(cid == 1)
        def _():
            lax.fori_loop(0, nb1, body, 0)
        plsc.subcore_barrier()
        pltpu.sync_copy(acc.at[pl.ds(sid * rpt, rpt)],
                        out_hbm.at[cid, pl.ds(sid * rpt, rpt)])

    return degk(dstp, ones_rows, zrows)


def _tc_matmul(x, w):
    n, k = x.shape
    d = w.shape[1]
    blk = 2000 if n % 2000 == 0 else n
    grid = n // blk

    def body(x_ref, w_ref, h_ref):
        h_ref[...] = jnp.dot(x_ref[...], w_ref[...],
                             preferred_element_type=jnp.float32)

    return pl.pallas_call(
        body,
        grid=(grid,),
        in_specs=[pl.BlockSpec((blk, k), lambda i: (i, 0)),
                  pl.BlockSpec((k, d), lambda i: (0, 0))],
        out_specs=pl.BlockSpec((blk, d), lambda i: (i, 0)),
        out_shape=jax.ShapeDtypeStruct((n, d), jnp.float32),
    )(x, w)


def _quant(g, md, dp):
    m = jnp.max(jnp.abs(g))[None, None]
    q = jnp.maximum(m * md, 1e-20) / 32000.0
    gq = jnp.floor(g / q + 0.5).astype(jnp.int16)
    pad = jnp.zeros((g.shape[0], dp - g.shape[1]), jnp.int16)
    return q, jnp.concatenate([gq, pad], axis=1)


def _dequant(spq_ref, q_ref, n, d):
    s = spq_ref[0].astype(jnp.float32) + spq_ref[1].astype(jnp.float32)
    return s[:n, :d] * q_ref[...]


def _tc_stage1(d0, d1, h, dp):
    n, d = h.shape

    def body(d0_ref, d1_ref, h_ref, dinv_ref, g_ref, gq_ref, q_ref, md_ref):
        deg = d0_ref[...] + d1_ref[...] + 1.0
        dinv = lax.rsqrt(deg)
        dinv_ref[...] = dinv
        g = dinv * h_ref[...]
        g_ref[...] = g
        md = jnp.max(deg)[None, None]
        md_ref[...] = md
        q, gq = _quant(g, md, dp)
        q_ref[...] = q
        gq_ref[...] = gq

    return pl.pallas_call(
        body,
        out_shape=[jax.ShapeDtypeStruct((n, 1), jnp.float32),
                   jax.ShapeDtypeStruct((n, d), jnp.float32),
                   jax.ShapeDtypeStruct((n, dp), jnp.int16),
                   jax.ShapeDtypeStruct((1, 1), jnp.float32),
                   jax.ShapeDtypeStruct((1, 1), jnp.float32)],
    )(d0, d1, h)


def _tc_combine_next(spq, q, md, gp, dinv, b, w, dp):
    n, d = gp.shape
    dn = w.shape[1]

    def body(spq_ref, q_ref, md_ref, gp_ref, dinv_ref, b_ref, w_ref,
             g_ref, gq_ref, qn_ref):
        s = _dequant(spq_ref, q_ref, n, d)
        xk = dinv_ref[...] * (s + gp_ref[...]) + b_ref[...]
        g = dinv_ref[...] * jnp.dot(xk, w_ref[...],
                                    preferred_element_type=jnp.float32)
        g_ref[...] = g
        qn, gq = _quant(g, md_ref[...], dp)
        qn_ref[...] = qn
        gq_ref[...] = gq

    return pl.pallas_call(
        body,
        out_shape=[jax.ShapeDtypeStruct((n, dn), jnp.float32),
                   jax.ShapeDtypeStruct((n, dp), jnp.int16),
                   jax.ShapeDtypeStruct((1, 1), jnp.float32)],
    )(spq, q, md, gp, dinv, b, w)


def _tc_scaled_x(spq, q, md, gp, dinv, b, dp):
    n, d = gp.shape

    def body(spq_ref, q_ref, md_ref, gp_ref, dinv_ref, b_ref,
             y_ref, yq_ref, qn_ref):
        s = _dequant(spq_ref, q_ref, n, d)
        dinv = dinv_ref[...]
        y = dinv * (dinv * (s + gp_ref[...]) + b_ref[...])
        y_ref[...] = y
        qn, yq = _quant(y, md_ref[...], dp)
        qn_ref[...] = qn
        yq_ref[...] = yq

    return pl.pallas_call(
        body,
        out_shape=[jax.ShapeDtypeStruct((n, d), jnp.float32),
                   jax.ShapeDtypeStruct((n, dp), jnp.int16),
                   jax.ShapeDtypeStruct((1, 1), jnp.float32)],
    )(spq, q, md, gp, dinv, b)


def _tc_combine_final(spq, q, yp, dinv, b, w):
    n, d = yp.shape
    dn = w.shape[1]

    def body(spq_ref, q_ref, yp_ref, dinv_ref, b_ref, w_ref, o_ref):
        u = _dequant(spq_ref, q_ref, n, d) + yp_ref[...]
        o_ref[...] = dinv_ref[...] * jnp.dot(
            u, w_ref[...], preferred_element_type=jnp.float32) + b_ref[...]

    return pl.pallas_call(
        body,
        out_shape=jax.ShapeDtypeStruct((n, dn), jnp.float32),
    )(spq, q, yp, dinv, b, w)


def kernel(x, edge_index, W1, b1, W2, b2, W3, b3):
    n = x.shape[0]
    e = edge_index.shape[1]

    nbt = -(-e // (N_TILE * B_EDGE * 8)) * 8
    nb0 = max(8, (int(nbt * 0.3) // 8) * 8)
    nb1 = nbt - nb0
    ep = N_TILE * nbt * B_EDGE
    dummy = n
    rpt = -(-(n + 1) // N_TILE)
    rpt = -(-rpt // 8) * 8
    nacc = N_TILE * rpt

    src = edge_index[0].astype(jnp.int32)
    dst = edge_index[1].astype(jnp.int32)

    def split_blocks(idx, fill):
        flat = jnp.concatenate(
            [idx, jnp.full((ep - e,), fill, jnp.int32)]).reshape(-1, B_EDGE)
        c0 = flat[:N_TILE * nb0].reshape(N_TILE, nb0, B_EDGE)
        c0 = jnp.pad(c0, ((0, 0), (0, max(nb0, nb1) - nb0), (0, 0)))
        c1 = flat[N_TILE * nb0:].reshape(N_TILE, nb1, B_EDGE)
        c1 = jnp.pad(c1, ((0, 0), (0, max(nb0, nb1) - nb1), (0, 0)))
        return jnp.concatenate([c0, c1], axis=0)

    srcp = split_blocks(src, 0)
    dstp = split_blocks(dst, dummy)

    ones_rows = jnp.ones((B_EDGE, DEG_W), jnp.float32)
    z_deg = jnp.zeros((rpt, DEG_W), jnp.float32)
    d_hid = W1.shape[1]
    d_out = W3.shape[1]
    dp_hid = -(-d_hid // 32) * 32
    z_hid = jnp.zeros((rpt, dp_hid), jnp.int16)

    degp = _sc_degree(dstp, ones_rows, z_deg, nacc, rpt, nb0, nb1)
    h1 = _tc_matmul(x, W1)
    dinv, g1, gq1, q1, md = _tc_stage1(degp[0, :n, 0:1], degp[1, :n, 0:1],
                                       h1, dp_hid)

    spq = _sc_segsum(gq1, srcp, dstp, z_hid, nacc, rpt, nb0, nb1)
    g2, gq2, q2 = _tc_combine_next(spq, q1, md, g1, dinv,
                                   b1.reshape(1, -1), W2, dp_hid)

    spq = _sc_segsum(gq2, srcp, dstp, z_hid, nacc, rpt, nb0, nb1)
    y, yq, q3 = _tc_scaled_x(spq, q2, md, g2, dinv, b2.reshape(1, -1), dp_hid)

    spq = _sc_segsum(yq, srcp, dstp, z_hid, nacc, rpt, nb0, nb1)
    return _tc_combine_final(spq, q3, y, dinv, b3.reshape(1, -1), W3)


# speedup vs baseline: 1.2021x; 1.2021x over previous
import functools

import jax
import jax.numpy as jnp
from jax import lax
from jax.experimental import pallas as pl
from jax.experimental.pallas import tpu as pltpu
from jax.experimental.pallas import tpu_sc as plsc

N_SC = 2
N_TILE = 16
NW = N_SC * N_TILE
B_EDGE = 128
DEG_W = 8


def _sc_mesh():
    return plsc.VectorSubcoreMesh(core_axis_name="c", subcore_axis_name="s")


def _sc_segsum(g, srcp, dstp, zrows, nacc, rpt, nb0, nb1):
    n_nodes, d = g.shape
    dt = g.dtype
    nb = srcp.shape[1]

    @functools.partial(
        pl.kernel,
        mesh=_sc_mesh(),
        compiler_params=pltpu.CompilerParams(use_tc_tiling_on_sc=False),
        out_type=jax.ShapeDtypeStruct((N_SC, nacc, d), dt),
        scratch_types=[
            pltpu.VMEM((nb, B_EDGE), jnp.int32),
            pltpu.VMEM((nb, B_EDGE), jnp.int32),
            [pltpu.VMEM((B_EDGE, d), dt)] * 8,
            pltpu.VMEM_SHARED((nacc, d), dt),
            [pltpu.SemaphoreType.DMA] * 8,
            [pltpu.SemaphoreType.DMA] * 8,
        ],
    )
    def seg(g_hbm, src_hbm, dst_hbm, z_hbm, out_hbm,
            src_v, dst_v, bufs, acc, gsems, ssems):
        cid = lax.axis_index("c")
        sid = lax.axis_index("s")
        wid = cid * N_TILE + sid

        pltpu.sync_copy(src_hbm.at[wid], src_v)
        pltpu.sync_copy(dst_hbm.at[wid], dst_v)
        pltpu.sync_copy(z_hbm, acc.at[pl.ds(sid * rpt, rpt)])
        plsc.subcore_barrier()

        def gather_start(j, b):
            pltpu.make_async_copy(g_hbm.at[src_v.at[j]], bufs[b], gsems[b]).start()

        def gather_wait(b):
            pltpu.make_async_copy(g_hbm.at[src_v.at[0]], bufs[b], gsems[b]).wait()

        def scatter_start(j, b):
            pltpu.make_async_copy(bufs[b], acc.at[dst_v.at[j]], ssems[b]).start(add=True)

        def scatter_wait(b):
            pltpu.make_async_copy(bufs[b], acc.at[dst_v.at[0]], ssems[b]).wait()

        nbuf = 8
        look = nbuf // 2

        def run_ring(nbc):
            for k in range(look):
                gather_start(k, k)

            def body(i, _):
                for b in range(nbuf):
                    j = nbuf * i + b
                    gather_wait(b)
                    scatter_start(j, b)
                    nxt = (b + look) % nbuf

                    @pl.when(j + look < nbc)
                    def _():
                        @pl.when(j >= look)
                        def _():
                            scatter_wait(nxt)

                        gather_start(j + look, nxt)
                return 0

            lax.fori_loop(0, nbc // nbuf, body, 0)
            for k in range(nbuf):
                scatter_wait(k)

        @pl.when(cid == 0)
        def _():
            run_ring(nb0)

        @pl.when(cid == 1)
        def _():
            run_ring(nb1)
        plsc.subcore_barrier()
        pltpu.sync_copy(acc.at[pl.ds(sid * rpt, rpt)],
                        out_hbm.at[cid, pl.ds(sid * rpt, rpt)])

    return seg(g, srcp, dstp, zrows)


def _sc_degree(dstp, ones_rows, zrows, nacc, rpt, nb0, nb1):
    nb = dstp.shape[1]

    @functools.partial(
        pl.kernel,
        mesh=_sc_mesh(),
        compiler_params=pltpu.CompilerParams(use_tc_tiling_on_sc=False),
        out_type=jax.ShapeDtypeStruct((N_SC, nacc, DEG_W), jnp.float32),
        scratch_types=[
            pltpu.VMEM((nb, B_EDGE), jnp.int32),
            pltpu.VMEM((B_EDGE, DEG_W), jnp.float32),
            pltpu.VMEM_SHARED((nacc, DEG_W), jnp.float32),
        ],
    )
    def degk(dst_hbm, ones_hbm, z_hbm, out_hbm, dst_v, ones_v, acc):
        cid = lax.axis_index("c")
        sid = lax.axis_index("s")
        wid = cid * N_TILE + sid

        pltpu.sync_copy(dst_hbm.at[wid], dst_v)
        pltpu.sync_copy(ones_hbm, ones_v)
        pltpu.sync_copy(z_hbm, acc.at[pl.ds(sid * rpt, rpt)])
        plsc.subcore_barrier()

        def body(j, _):
            pltpu.sync_copy(ones_v, acc.at[dst_v.at[j]], add=True)
            return 0

        @pl.when(cid == 0)
        def _():
            lax.fori_loop(0, nb0, body, 0)

        @pl.when(cid == 1)
        def _():
            lax.fori_loop(0, nb1, body, 0)
        plsc.subcore_barrier()
        pltpu.sync_copy(acc.at[pl.ds(sid * rpt, rpt)],
                        out_hbm.at[cid, pl.ds(sid * rpt, rpt)])

    return degk(dstp, ones_rows, zrows)


def _tc_matmul(x, w):
    n, k = x.shape
    d = w.shape[1]
    blk = 2000 if n % 2000 == 0 else n
    grid = n // blk

    def body(x_ref, w_ref, h_ref):
        h_ref[...] = jnp.dot(x_ref[...], w_ref[...],
                             preferred_element_type=jnp.float32)

    return pl.pallas_call(
        body,
        grid=(grid,),
        in_specs=[pl.BlockSpec((blk, k), lambda i: (i, 0)),
                  pl.BlockSpec((k, d), lambda i: (0, 0))],
        out_specs=pl.BlockSpec((blk, d), lambda i: (i, 0)),
        out_shape=jax.ShapeDtypeStruct((n, d), jnp.float32),
    )(x, w)


def _quant(g, md, dp):
    m = jnp.max(jnp.abs(g))[None, None]
    q = jnp.maximum(m * md, 1e-20) / 32000.0
    gq = jnp.floor(g / q + 0.5).astype(jnp.int16)
    pad = jnp.zeros((g.shape[0], dp - g.shape[1]), jnp.int16)
    return q, jnp.concatenate([gq, pad], axis=1)


def _dequant(spq_ref, q_ref, n, d):
    s = spq_ref[0].astype(jnp.float32) + spq_ref[1].astype(jnp.float32)
    return s[:n, :d] * q_ref[...]


def _tc_stage1(d0, d1, h, dp):
    n, d = h.shape

    def body(d0_ref, d1_ref, h_ref, dinv_ref, g_ref, gq_ref, q_ref, md_ref):
        deg = d0_ref[...] + d1_ref[...] + 1.0
        dinv = lax.rsqrt(deg)
        dinv_ref[...] = dinv
        g = dinv * h_ref[...]
        g_ref[...] = g
        md = jnp.max(deg)[None, None]
        md_ref[...] = md
        q, gq = _quant(g, md, dp)
        q_ref[...] = q
        gq_ref[...] = gq

    return pl.pallas_call(
        body,
        out_shape=[jax.ShapeDtypeStruct((n, 1), jnp.float32),
                   jax.ShapeDtypeStruct((n, d), jnp.float32),
                   jax.ShapeDtypeStruct((n, dp), jnp.int16),
                   jax.ShapeDtypeStruct((1, 1), jnp.float32),
                   jax.ShapeDtypeStruct((1, 1), jnp.float32)],
    )(d0, d1, h)


def _tc_combine_next(spq, q, md, gp, dinv, b, w, dp):
    n, d = gp.shape
    dn = w.shape[1]

    def body(spq_ref, q_ref, md_ref, gp_ref, dinv_ref, b_ref, w_ref,
             g_ref, gq_ref, qn_ref):
        s = _dequant(spq_ref, q_ref, n, d)
        xk = dinv_ref[...] * (s + gp_ref[...]) + b_ref[...]
        g = dinv_ref[...] * jnp.dot(xk, w_ref[...],
                                    preferred_element_type=jnp.float32)
        g_ref[...] = g
        qn, gq = _quant(g, md_ref[...], dp)
        qn_ref[...] = qn
        gq_ref[...] = gq

    return pl.pallas_call(
        body,
        out_shape=[jax.ShapeDtypeStruct((n, dn), jnp.float32),
                   jax.ShapeDtypeStruct((n, dp), jnp.int16),
                   jax.ShapeDtypeStruct((1, 1), jnp.float32)],
    )(spq, q, md, gp, dinv, b, w)


def _tc_scaled_x(spq, q, md, gp, dinv, b, dp):
    n, d = gp.shape

    def body(spq_ref, q_ref, md_ref, gp_ref, dinv_ref, b_ref,
             y_ref, yq_ref, qn_ref):
        s = _dequant(spq_ref, q_ref, n, d)
        dinv = dinv_ref[...]
        y = dinv * (dinv * (s + gp_ref[...]) + b_ref[...])
        y_ref[...] = y
        qn, yq = _quant(y, md_ref[...], dp)
        qn_ref[...] = qn
        yq_ref[...] = yq

    return pl.pallas_call(
        body,
        out_shape=[jax.ShapeDtypeStruct((n, d), jnp.float32),
                   jax.ShapeDtypeStruct((n, dp), jnp.int16),
                   jax.ShapeDtypeStruct((1, 1), jnp.float32)],
    )(spq, q, md, gp, dinv, b)


def _tc_combine_final(spq, q, yp, dinv, b, w):
    n, d = yp.shape
    dn = w.shape[1]

    def body(spq_ref, q_ref, yp_ref, dinv_ref, b_ref, w_ref, o_ref):
        u = _dequant(spq_ref, q_ref, n, d) + yp_ref[...]
        o_ref[...] = dinv_ref[...] * jnp.dot(
            u, w_ref[...], preferred_element_type=jnp.float32) + b_ref[...]

    return pl.pallas_call(
        body,
        out_shape=jax.ShapeDtypeStruct((n, dn), jnp.float32),
    )(spq, q, yp, dinv, b, w)


def kernel(x, edge_index, W1, b1, W2, b2, W3, b3):
    n = x.shape[0]
    e = edge_index.shape[1]

    nbt = -(-e // (N_TILE * B_EDGE * 8)) * 8
    nb1 = max(8, (int(nbt * 0.3) // 8) * 8)
    nb0 = nbt - nb1
    ep = N_TILE * nbt * B_EDGE
    dummy = n
    rpt = -(-(n + 1) // N_TILE)
    rpt = -(-rpt // 8) * 8
    nacc = N_TILE * rpt

    src = edge_index[0].astype(jnp.int32)
    dst = edge_index[1].astype(jnp.int32)

    def split_blocks(idx, fill):
        flat = jnp.concatenate(
            [idx, jnp.full((ep - e,), fill, jnp.int32)]).reshape(-1, B_EDGE)
        c0 = flat[:N_TILE * nb0].reshape(N_TILE, nb0, B_EDGE)
        c0 = jnp.pad(c0, ((0, 0), (0, max(nb0, nb1) - nb0), (0, 0)))
        c1 = flat[N_TILE * nb0:].reshape(N_TILE, nb1, B_EDGE)
        c1 = jnp.pad(c1, ((0, 0), (0, max(nb0, nb1) - nb1), (0, 0)))
        return jnp.concatenate([c0, c1], axis=0)

    srcp = split_blocks(src, 0)
    dstp = split_blocks(dst, dummy)

    ones_rows = jnp.ones((B_EDGE, DEG_W), jnp.float32)
    z_deg = jnp.zeros((rpt, DEG_W), jnp.float32)
    d_hid = W1.shape[1]
    d_out = W3.shape[1]
    dp_hid = -(-d_hid // 32) * 32
    z_hid = jnp.zeros((rpt, dp_hid), jnp.int16)

    degp = _sc_degree(dstp, ones_rows, z_deg, nacc, rpt, nb0, nb1)
    h1 = _tc_matmul(x, W1)
    dinv, g1, gq1, q1, md = _tc_stage1(degp[0, :n, 0:1], degp[1, :n, 0:1],
                                       h1, dp_hid)

    spq = _sc_segsum(gq1, srcp, dstp, z_hid, nacc, rpt, nb0, nb1)
    g2, gq2, q2 = _tc_combine_next(spq, q1, md, g1, dinv,
                                   b1.reshape(1, -1), W2, dp_hid)

    spq = _sc_segsum(gq2, srcp, dstp, z_hid, nacc, rpt, nb0, nb1)
    y, yq, q3 = _tc_scaled_x(spq, q2, md, g2, dinv, b2.reshape(1, -1), dp_hid)

    spq = _sc_segsum(yq, srcp, dstp, z_hid, nacc, rpt, nb0, nb1)
    return _tc_combine_final(spq, q3, y, dinv, b3.reshape(1, -1), W3)
